# baseline (speedup 1.0000x reference)
import functools

import jax
import jax.numpy as jnp
from jax import lax
from jax.experimental import pallas as pl
from jax.experimental.pallas import tpu as pltpu
from jax.experimental.pallas import tpu_sc as plsc

_info = plsc.get_sparse_core_info()
_NC, _NS = _info.num_cores, _info.num_subcores
_NW = _NC * _NS

_CHUNK = 128


@functools.partial(jax.jit, static_argnames=("chunks",))
def _sc_gather(weight, idx3, *, chunks):
    total = _NW * chunks * _CHUNK
    dim = weight.shape[1]
    per_w = chunks * _CHUNK

    mesh = plsc.VectorSubcoreMesh(core_axis_name="c", subcore_axis_name="s")

    @functools.partial(
        pl.kernel,
        out_type=jax.ShapeDtypeStruct((total, dim), jnp.float32),
        mesh=mesh,
        scratch_types=[
            pltpu.VMEM((chunks, _CHUNK), jnp.int32),
            pltpu.VMEM((_CHUNK, dim), jnp.float32),
            pltpu.SemaphoreType.DMA,
        ],
        compiler_params=pltpu.CompilerParams(use_tc_tiling_on_sc=False),
    )
    def k(table_hbm, idx_hbm, out_hbm, idx_v, rows_v, sem):
        wid = lax.axis_index("s") * _NC + lax.axis_index("c")
        base = wid * per_w
        pltpu.sync_copy(idx_hbm.at[wid], idx_v)

        @pl.loop(0, chunks)
        def _(j):
            pltpu.async_copy(table_hbm.at[idx_v.at[j]], rows_v, sem).wait()
            pltpu.sync_copy(rows_v, out_hbm.at[pl.ds(base + j * _CHUNK, _CHUNK)])

    return k(weight, idx3)


def kernel(idx, weight):
    b, s = idx.shape
    total = b * s
    per_w = total // _NW
    chunks = per_w // _CHUNK
    idx3 = idx.astype(jnp.int32).reshape(_NW, chunks, _CHUNK)
    out = _sc_gather(weight, idx3, chunks=chunks)
    return out.reshape(b, s, weight.shape[1])


# speedup vs baseline: 1.0443x; 1.0443x over previous
import functools

import jax
import jax.numpy as jnp
from jax import lax
from jax.experimental import pallas as pl
from jax.experimental.pallas import tpu as pltpu
from jax.experimental.pallas import tpu_sc as plsc

_info = plsc.get_sparse_core_info()
_NC, _NS = _info.num_cores, _info.num_subcores
_NW = _NC * _NS

_CHUNK = 128


_K = 5


@functools.partial(jax.jit, static_argnames=("chunks",))
def _sc_gather(weight, idx3, *, chunks):
    total = _NW * chunks * _CHUNK
    dim = weight.shape[1]
    per_w = chunks * _CHUNK
    groups = chunks // _K

    mesh = plsc.VectorSubcoreMesh(core_axis_name="c", subcore_axis_name="s")

    @functools.partial(
        pl.kernel,
        out_type=jax.ShapeDtypeStruct((total, dim), jnp.float32),
        mesh=mesh,
        scratch_types=[
            pltpu.VMEM((chunks, _CHUNK), jnp.int32),
            pltpu.VMEM((2, _K, _CHUNK, dim), jnp.float32),
            pltpu.SemaphoreType.DMA,
            pltpu.SemaphoreType.DMA,
        ],
        compiler_params=pltpu.CompilerParams(use_tc_tiling_on_sc=False),
    )
    def k(table_hbm, idx_hbm, out_hbm, idx_v, rows_v, gsem, wsem):
        wid = lax.axis_index("s") * _NC + lax.axis_index("c")
        base = wid * per_w
        pltpu.sync_copy(idx_hbm.at[wid], idx_v)

        def fire_gathers(g, p):
            for b in range(_K):
                pltpu.async_copy(
                    table_hbm.at[idx_v.at[g * _K + b]], rows_v.at[p, b], gsem
                )

        def drain_gathers_fire_writes(g, p):
            for b in range(_K):
                pltpu.make_async_copy(
                    table_hbm.at[idx_v.at[g * _K + b]], rows_v.at[p, b], gsem
                ).wait()
                pltpu.async_copy(
                    rows_v.at[p, b],
                    out_hbm.at[pl.ds(base + (g * _K + b) * _CHUNK, _CHUNK)],
                    wsem,
                )

        def drain_writes(g, p):
            for b in range(_K):
                pltpu.make_async_copy(
                    rows_v.at[p, b],
                    out_hbm.at[pl.ds(base + (g * _K + b) * _CHUNK, _CHUNK)],
                    wsem,
                ).wait()

        fire_gathers(0, 0)
        drain_gathers_fire_writes(0, 0)
        fire_gathers(1, 1)

        @pl.loop(1, groups - 1)
        def _(g):
            p = lax.rem(g, 2)
            drain_writes(g - 1, 1 - p)
            drain_gathers_fire_writes(g, p)
            fire_gathers(g + 1, 1 - p)

        gl = groups - 1
        pl_ = gl % 2
        drain_writes(gl - 1, 1 - pl_)
        drain_gathers_fire_writes(gl, pl_)
        drain_writes(gl, pl_)

    return k(weight, idx3)


def kernel(idx, weight):
    b, s = idx.shape
    total = b * s
    per_w = total // _NW
    chunks = per_w // _CHUNK
    idx3 = idx.astype(jnp.int32).reshape(_NW, chunks, _CHUNK)
    out = _sc_gather(weight, idx3, chunks=chunks)
    return out.reshape(b, s, weight.shape[1])


# speedup vs baseline: 1.3457x; 1.2886x over previous
import functools

import jax
import jax.numpy as jnp
from jax import lax
from jax.experimental import pallas as pl
from jax.experimental.pallas import tpu as pltpu
from jax.experimental.pallas import tpu_sc as plsc

_info = plsc.get_sparse_core_info()
_NC, _NS = _info.num_cores, _info.num_subcores
_NW = _NC * _NS

_C = 64


@functools.partial(jax.jit, static_argnames=("chunks",))
def _sc_gather(weight, idxf, *, chunks):
    total = _NW * chunks * _C
    dim = weight.shape[1]
    per_w = chunks * _C

    mesh = plsc.VectorSubcoreMesh(core_axis_name="c", subcore_axis_name="s")

    @functools.partial(
        pl.kernel,
        out_type=jax.ShapeDtypeStruct((total, dim), jnp.float32),
        mesh=mesh,
        scratch_types=[
            pltpu.VMEM((per_w,), jnp.int32),
            pltpu.VMEM((2, _C, dim), jnp.float32),
            pltpu.SemaphoreType.DMA((2,)),
            pltpu.SemaphoreType.DMA((2,)),
        ],
        compiler_params=pltpu.CompilerParams(use_tc_tiling_on_sc=True),
    )
    def k(table_hbm, idx_hbm, out_hbm, idx_v, rows_v, gsem, wsem):
        wid = lax.axis_index("s") * _NC + lax.axis_index("c")
        base = wid * per_w

        pltpu.sync_copy(idx_hbm.at[pl.ds(base, per_w)], idx_v)

        def fire_rows(j, p):
            for g in range(_C // 16):
                v = idx_v[pl.ds(j * _C + g * 16, 16)]
                for r in range(16):
                    iv = v[r]
                    pltpu.async_copy(
                        table_hbm.at[pl.ds(iv, 1)],
                        rows_v.at[p].at[pl.ds(g * 16 + r, 1)],
                        gsem.at[p],
                    )

        def drain_rows(p):
            for r in range(_C):
                pltpu.make_async_copy(
                    table_hbm.at[pl.ds(0, 1)],
                    rows_v.at[p].at[pl.ds(r, 1)],
                    gsem.at[p],
                ).wait()

        def fire_write(j, p):
            pltpu.async_copy(
                rows_v.at[p], out_hbm.at[pl.ds(base + j * _C, _C)], wsem.at[p]
            )

        def drain_write(j, p):
            pltpu.make_async_copy(
                rows_v.at[p], out_hbm.at[pl.ds(base + j * _C, _C)], wsem.at[p]
            ).wait()

        fire_rows(0, 0)

        @pl.loop(1, chunks)
        def _(j):
            p = lax.rem(j, 2)
            @pl.when(j >= 2)
            def _():
                drain_write(j - 2, p)
            fire_rows(j, p)
            drain_rows(1 - p)
            fire_write(j - 1, 1 - p)

        pf = lax.rem(chunks - 1, 2)
        drain_rows(pf)
        fire_write(chunks - 1, pf)
        drain_write(chunks - 2, 1 - pf)
        drain_write(chunks - 1, pf)

    return k(weight, idxf)


def kernel(idx, weight):
    b, s = idx.shape
    total = b * s
    per_w = total // _NW
    chunks = per_w // _C
    idxf = idx.astype(jnp.int32).reshape(-1)
    out = _sc_gather(weight, idxf, chunks=chunks)
    return out.reshape(b, s, weight.shape[1])


# speedup vs baseline: 1.7092x; 1.2701x over previous
import functools

import jax
import jax.numpy as jnp
from jax import lax
from jax.experimental import pallas as pl
from jax.experimental.pallas import tpu as pltpu
from jax.experimental.pallas import tpu_sc as plsc

_info = plsc.get_sparse_core_info()
_NC, _NS = _info.num_cores, _info.num_subcores
_NW = _NC * _NS


@jax.jit
def _sc_gather(weight, idx):
    nb, sl = idx.shape
    dim = weight.shape[1]
    per_w = nb // _NW

    mesh = plsc.VectorSubcoreMesh(core_axis_name="c", subcore_axis_name="s")

    @functools.partial(
        pl.kernel,
        out_type=jax.ShapeDtypeStruct((nb, sl, dim), jnp.float32),
        mesh=mesh,
        scratch_types=[
            pltpu.VMEM((2, 1, 50), jnp.float32),
            pltpu.VMEM((2, 1, sl, dim), jnp.float32),
            pltpu.SemaphoreType.DMA((2,)),
            pltpu.SemaphoreType.DMA((2,)),
            pltpu.SemaphoreType.DMA((2,)),
        ],
        compiler_params=pltpu.CompilerParams(use_tc_tiling_on_sc=True, needs_layout_passes=False),
    )
    def k(table_hbm, idx_hbm, out_hbm, idxb, rows_v, isem, gsem, wsem):
        wid = lax.axis_index("s") * _NC + lax.axis_index("c")
        base = wid * per_w

        def load_idx(j, p):
            pltpu.async_copy(
                idx_hbm.at[pl.ds(base + j, 1)],
                idxb.at[p],
                isem.at[p],
            )

        def wait_idx(j, p):
            pltpu.make_async_copy(
                idx_hbm.at[pl.ds(base + j, 1)],
                idxb.at[p],
                isem.at[p],
            ).wait()

        def fire_rows(j, p):
            vecs = [
                plsc.bitcast(idxb.at[p, 0][pl.ds(o, 16)], jnp.int32)
                for o in (0, 16, 32, 34)
            ]
            for s in range(sl):
                g, l = divmod(s, 16)
                if s >= 48:
                    g, l = 3, s - 34
                iv = vecs[g][l]
                pltpu.async_copy(
                    table_hbm.at[pl.ds(iv, 1)],
                    rows_v.at[p, 0].at[pl.ds(s, 1)],
                    gsem.at[p],
                )

        def drain_rows(p):
            for s in range(sl):
                pltpu.make_async_copy(
                    table_hbm.at[pl.ds(0, 1)], rows_v.at[p, 0].at[pl.ds(s, 1)], gsem.at[p]
                ).wait()

        def fire_write(j, p):
            pltpu.async_copy(rows_v.at[p], out_hbm.at[pl.ds(base + j, 1)], wsem.at[p])

        def drain_write(j, p):
            pltpu.make_async_copy(
                rows_v.at[p], out_hbm.at[pl.ds(base + j, 1)], wsem.at[p]
            ).wait()

        load_idx(0, 0)
        wait_idx(0, 0)
        fire_rows(0, 0)
        load_idx(1, 1)

        @pl.loop(1, per_w)
        def _(j):
            p = lax.rem(j, 2)
            wait_idx(j, p)
            @pl.when(j >= 2)
            def _():
                drain_write(j - 2, p)
            fire_rows(j, p)
            @pl.when(j + 1 < per_w)
            def _():
                load_idx(j + 1, 1 - p)
            drain_rows(1 - p)
            fire_write(j - 1, 1 - p)

        pf = lax.rem(per_w - 1, 2)
        drain_rows(pf)
        fire_write(per_w - 1, pf)
        drain_write(per_w - 2, 1 - pf)
        drain_write(per_w - 1, pf)

    return k(weight, jax.lax.bitcast_convert_type(idx, jnp.float32))


def kernel(idx, weight):
    return _sc_gather(weight, idx)
